# baseline (speedup 1.0000x reference)
import functools

import jax
import jax.numpy as jnp
from jax import lax
from jax.experimental import pallas as pl
from jax.experimental.pallas import tpu as pltpu
from jax.experimental.pallas import tpu_sc as plsc

D_MODEL = 1024
_NUM_CORES = 2
_NUM_SUBCORES = 16
_NUM_WORKERS = _NUM_CORES * _NUM_SUBCORES


@functools.lru_cache(maxsize=None)
def _build_mpmd_kernel(
    nb: int,
    seq: int,
    tec_cols: int,
    chunk: int,
    nbuf: int,
    lag: int,
    s_chunk: int,
):
    B = nb * seq
    scs_cols = seq - tec_cols
    workers_per_row = _NUM_WORKERS // nb
    rows_per_worker = tec_cols // workers_per_row
    n_chunks = rows_per_worker // chunk
    rows_per_scs = (nb // _NUM_CORES) * scs_cols
    n_s_chunks = rows_per_scs // s_chunk
    s_chunks_per_row = scs_cols // s_chunk

    vmesh = plsc.VectorSubcoreMesh(core_axis_name="c", subcore_axis_name="s")
    smesh = plsc.ScalarSubcoreMesh(axis_name="c", num_cores=_NUM_CORES)

    def tec_fn(toks_hbm, table_hbm, out_hbm, idx_v, buf_v, gsems, osems,
               idx_s, buf_s, s_gsem, s_osem):
        del idx_s, buf_s, s_gsem, s_osem
        wid = lax.axis_index("s") * _NUM_CORES + lax.axis_index("c")
        brow = wid % nb
        col = (wid // nb) * rows_per_worker
        base = brow * seq + col
        pltpu.sync_copy(toks_hbm.at[pl.ds(base, rows_per_worker)], idx_v)

        def gather_copy(ci, slot):
            return pltpu.make_async_copy(
                table_hbm.at[idx_v.at[pl.ds(ci * chunk, chunk)]],
                buf_v.at[slot],
                gsems[slot],
            )

        def out_copy(ci, slot):
            return pltpu.make_async_copy(
                buf_v.at[slot],
                out_hbm.at[pl.ds(base + ci * chunk, chunk)],
                osems[slot],
            )

        for i in range(n_chunks + lag):
            if i < n_chunks:
                s = i % nbuf
                if i >= nbuf:
                    out_copy(i - nbuf, s).wait()
                gather_copy(i, s).start()
            j = i - lag
            if j >= 0:
                gather_copy(j, j % nbuf).wait()
                out_copy(j, j % nbuf).start()
        for j in range(max(0, n_chunks - nbuf), n_chunks):
            out_copy(j, j % nbuf).wait()

    def scs_fn(toks_hbm, table_hbm, out_hbm, idx_v, buf_v, gsems, osems,
               idx_s, buf_s, s_gsem, s_osem):
        del idx_v, buf_v, gsems, osems
        cid = lax.axis_index("c")

        def chunk_pos(ci):
            brow = (nb // _NUM_CORES) * cid + ci // s_chunks_per_row
            col = tec_cols + (ci % s_chunks_per_row) * s_chunk
            return brow, col

        def out_copy(ci, slot):
            brow, col = chunk_pos(ci)
            return pltpu.make_async_copy(
                buf_s.at[slot],
                out_hbm.at[pl.ds(brow * seq + col, s_chunk)],
                s_osem,
            )

        for ci in range(n_s_chunks):
            slot = ci % 2
            brow, col = chunk_pos(ci)
            if ci >= 2:
                out_copy(ci - 2, slot).wait()
            pltpu.sync_copy(toks_hbm.at[pl.ds(brow * seq + col, s_chunk)], idx_s)

            def row_body(r, slot=slot):
                pltpu.make_async_copy(
                    table_hbm.at[pl.ds(idx_s[r], 1)],
                    buf_s.at[slot].at[pl.ds(r, 1)],
                    s_gsem,
                ).start()

            pl.loop(0, s_chunk)(row_body)
            pltpu.make_async_copy(
                table_hbm.at[pl.ds(0, s_chunk)], buf_s.at[slot], s_gsem
            ).wait()
            out_copy(ci, slot).start()
        for ci in range(max(0, n_s_chunks - 2), n_s_chunks):
            out_copy(ci, ci % 2).wait()

    return pl.kernel(
        body=[tec_fn, scs_fn],
        mesh=[vmesh, smesh],
        out_type=jax.ShapeDtypeStruct((B, D_MODEL), jnp.float32),
        scratch_types=[
            (pltpu.VMEM @ vmesh)((rows_per_worker,), jnp.int32),
            (pltpu.VMEM @ vmesh)((nbuf, chunk, D_MODEL), jnp.float32),
            tuple((pltpu.SemaphoreType.DMA @ vmesh) for _ in range(nbuf)),
            tuple((pltpu.SemaphoreType.DMA @ vmesh) for _ in range(nbuf)),
            (pltpu.SMEM @ smesh)((s_chunk,), jnp.int32),
            pltpu.VMEM_SHARED((2, s_chunk, D_MODEL), jnp.float32),
            pltpu.SemaphoreType.DMA @ smesh,
            pltpu.SemaphoreType.DMA @ smesh,
        ],
    )


def kernel(toks, W_E):
    n_batch, seq = toks.shape
    flat = toks.reshape(n_batch * seq).astype(jnp.int32)
    out = _build_mpmd_kernel(
        n_batch, seq, tec_cols=3 * seq // 4,
        chunk=16, nbuf=3, lag=2, s_chunk=512,
    )(flat, W_E)
    return out.reshape(n_batch, seq, D_MODEL)


# speedup vs baseline: 1.0049x; 1.0049x over previous
import functools

import jax
import jax.numpy as jnp
from jax import lax
from jax.experimental import pallas as pl
from jax.experimental.pallas import tpu as pltpu
from jax.experimental.pallas import tpu_sc as plsc

D_MODEL = 1024
_NUM_CORES = 2
_NUM_SUBCORES = 16
_NUM_WORKERS = _NUM_CORES * _NUM_SUBCORES


@functools.lru_cache(maxsize=None)
def _build_mpmd_kernel(
    nb: int,
    seq: int,
    tec_cols: int,
    chunk: int,
    nbuf: int,
    lag: int,
    s_chunk: int,
):
    B = nb * seq
    scs_cols = seq - tec_cols
    workers_per_row = _NUM_WORKERS // nb
    rows_per_worker = tec_cols // workers_per_row
    n_chunks = rows_per_worker // chunk
    rows_per_scs = (nb // _NUM_CORES) * scs_cols
    n_s_chunks = rows_per_scs // s_chunk
    s_chunks_per_row = scs_cols // s_chunk

    vmesh = plsc.VectorSubcoreMesh(core_axis_name="c", subcore_axis_name="s")
    smesh = plsc.ScalarSubcoreMesh(axis_name="c", num_cores=_NUM_CORES)

    def tec_fn(toks_hbm, table_hbm, out_hbm, idx_v, buf_v, gsems, osems,
               idx_s, buf_s, s_gsem, s_osem):
        del idx_s, buf_s, s_gsem, s_osem
        wid = lax.axis_index("s") * _NUM_CORES + lax.axis_index("c")
        brow = wid % nb
        col = (wid // nb) * rows_per_worker
        base = brow * seq + col
        pltpu.sync_copy(toks_hbm.at[pl.ds(base, rows_per_worker)], idx_v)

        def gather_copy(ci, slot):
            return pltpu.make_async_copy(
                table_hbm.at[idx_v.at[pl.ds(ci * chunk, chunk)]],
                buf_v.at[slot],
                gsems[slot],
            )

        def out_copy(ci, slot):
            return pltpu.make_async_copy(
                buf_v.at[slot],
                out_hbm.at[pl.ds(base + ci * chunk, chunk)],
                osems[slot],
            )

        for i in range(n_chunks + lag):
            if i < n_chunks:
                s = i % nbuf
                if i >= nbuf:
                    out_copy(i - nbuf, s).wait()
                gather_copy(i, s).start()
            j = i - lag
            if j >= 0:
                gather_copy(j, j % nbuf).wait()
                out_copy(j, j % nbuf).start()
        for j in range(max(0, n_chunks - nbuf), n_chunks):
            out_copy(j, j % nbuf).wait()

    def scs_fn(toks_hbm, table_hbm, out_hbm, idx_v, buf_v, gsems, osems,
               idx_s, buf_s, s_gsem, s_osem):
        del idx_v, buf_v, gsems, osems
        cid = lax.axis_index("c")

        def chunk_pos(ci):
            brow = (nb // _NUM_CORES) * cid + ci // s_chunks_per_row
            col = tec_cols + (ci % s_chunks_per_row) * s_chunk
            return brow, col

        def out_copy(ci, slot):
            brow, col = chunk_pos(ci)
            return pltpu.make_async_copy(
                buf_s.at[slot],
                out_hbm.at[pl.ds(brow * seq + col, s_chunk)],
                s_osem,
            )

        for ci in range(n_s_chunks):
            slot = ci % 2
            brow, col = chunk_pos(ci)
            if ci >= 2:
                out_copy(ci - 2, slot).wait()
            pltpu.sync_copy(toks_hbm.at[pl.ds(brow * seq + col, s_chunk)], idx_s)

            def row_body(r, slot=slot):
                pltpu.make_async_copy(
                    table_hbm.at[pl.ds(idx_s[r], 1)],
                    buf_s.at[slot].at[pl.ds(r, 1)],
                    s_gsem,
                ).start()

            pl.loop(0, s_chunk)(row_body)
            pltpu.make_async_copy(
                table_hbm.at[pl.ds(0, s_chunk)], buf_s.at[slot], s_gsem
            ).wait()
            out_copy(ci, slot).start()
        for ci in range(max(0, n_s_chunks - 2), n_s_chunks):
            out_copy(ci, ci % 2).wait()

    return pl.kernel(
        body=[tec_fn, scs_fn],
        mesh=[vmesh, smesh],
        out_type=jax.ShapeDtypeStruct((B, D_MODEL), jnp.float32),
        scratch_types=[
            (pltpu.VMEM @ vmesh)((rows_per_worker,), jnp.int32),
            (pltpu.VMEM @ vmesh)((nbuf, chunk, D_MODEL), jnp.float32),
            tuple((pltpu.SemaphoreType.DMA @ vmesh) for _ in range(nbuf)),
            tuple((pltpu.SemaphoreType.DMA @ vmesh) for _ in range(nbuf)),
            (pltpu.SMEM @ smesh)((s_chunk,), jnp.int32),
            pltpu.VMEM_SHARED((2, s_chunk, D_MODEL), jnp.float32),
            pltpu.SemaphoreType.DMA @ smesh,
            pltpu.SemaphoreType.DMA @ smesh,
        ],
    )


def kernel(toks, W_E):
    n_batch, seq = toks.shape
    flat = toks.reshape(n_batch * seq).astype(jnp.int32)
    out = _build_mpmd_kernel(
        n_batch, seq, tec_cols=3 * seq // 4,
        chunk=16, nbuf=4, lag=2, s_chunk=256,
    )(flat, W_E)
    return out.reshape(n_batch, seq, D_MODEL)


# speedup vs baseline: 1.0078x; 1.0029x over previous
import functools

import jax
import jax.numpy as jnp
from jax import lax
from jax.experimental import pallas as pl
from jax.experimental.pallas import tpu as pltpu
from jax.experimental.pallas import tpu_sc as plsc

D_MODEL = 1024
_NUM_CORES = 2
_NUM_SUBCORES = 16
_NUM_WORKERS = _NUM_CORES * _NUM_SUBCORES


@functools.lru_cache(maxsize=None)
def _build_mpmd_kernel(
    nb: int,
    seq: int,
    tec_cols: int,
    chunk: int,
    nbuf: int,
    lag: int,
    s_chunk: int,
):
    B = nb * seq
    scs_cols = seq - tec_cols
    workers_per_row = _NUM_WORKERS // nb
    rows_per_worker = tec_cols // workers_per_row
    n_chunks = rows_per_worker // chunk
    rows_per_scs = (nb // _NUM_CORES) * scs_cols
    n_s_chunks = rows_per_scs // s_chunk
    s_chunks_per_row = scs_cols // s_chunk

    vmesh = plsc.VectorSubcoreMesh(core_axis_name="c", subcore_axis_name="s")
    smesh = plsc.ScalarSubcoreMesh(axis_name="c", num_cores=_NUM_CORES)

    def tec_fn(toks_hbm, table_hbm, out_hbm, idx_v, buf_v, gsems, osems,
               idx_s, buf_s, s_gsem, s_osem):
        del idx_s, buf_s, s_gsem, s_osem
        wid = lax.axis_index("s") * _NUM_CORES + lax.axis_index("c")
        brow = wid % nb
        col = (wid // nb) * rows_per_worker
        base = brow * seq + col
        pltpu.sync_copy(toks_hbm.at[pl.ds(base, rows_per_worker)], idx_v)

        def gather_copy(ci, slot):
            return pltpu.make_async_copy(
                table_hbm.at[idx_v.at[pl.ds(ci * chunk, chunk)]],
                buf_v.at[slot],
                gsems[slot],
            )

        def out_copy(ci, slot):
            return pltpu.make_async_copy(
                buf_v.at[slot],
                out_hbm.at[pl.ds(base + ci * chunk, chunk)],
                osems[slot],
            )

        for i in range(n_chunks + lag):
            if i < n_chunks:
                s = i % nbuf
                if i >= nbuf:
                    out_copy(i - nbuf, s).wait()
                gather_copy(i, s).start()
            j = i - lag
            if j >= 0:
                gather_copy(j, j % nbuf).wait()
                out_copy(j, j % nbuf).start()
        for j in range(max(0, n_chunks - nbuf), n_chunks):
            out_copy(j, j % nbuf).wait()

    def scs_fn(toks_hbm, table_hbm, out_hbm, idx_v, buf_v, gsems, osems,
               idx_s, buf_s, s_gsem, s_osem):
        del idx_v, buf_v, gsems, osems
        cid = lax.axis_index("c")

        def chunk_pos(ci):
            brow = (nb // _NUM_CORES) * cid + ci // s_chunks_per_row
            col = tec_cols + (ci % s_chunks_per_row) * s_chunk
            return brow, col

        def out_copy(ci, slot):
            brow, col = chunk_pos(ci)
            return pltpu.make_async_copy(
                buf_s.at[slot],
                out_hbm.at[pl.ds(brow * seq + col, s_chunk)],
                s_osem,
            )

        for ci in range(n_s_chunks):
            slot = ci % 2
            brow, col = chunk_pos(ci)
            if ci >= 2:
                out_copy(ci - 2, slot).wait()
            pltpu.sync_copy(toks_hbm.at[pl.ds(brow * seq + col, s_chunk)], idx_s)

            def row_body(r, slot=slot):
                pltpu.make_async_copy(
                    table_hbm.at[pl.ds(idx_s[r], 1)],
                    buf_s.at[slot].at[pl.ds(r, 1)],
                    s_gsem,
                ).start()

            pl.loop(0, s_chunk)(row_body)
            pltpu.make_async_copy(
                table_hbm.at[pl.ds(0, s_chunk)], buf_s.at[slot], s_gsem
            ).wait()
            out_copy(ci, slot).start()
        for ci in range(max(0, n_s_chunks - 2), n_s_chunks):
            out_copy(ci, ci % 2).wait()

    return pl.kernel(
        body=[tec_fn, scs_fn],
        mesh=[vmesh, smesh],
        out_type=jax.ShapeDtypeStruct((B, D_MODEL), jnp.float32),
        scratch_types=[
            (pltpu.VMEM @ vmesh)((rows_per_worker,), jnp.int32),
            (pltpu.VMEM @ vmesh)((nbuf, chunk, D_MODEL), jnp.float32),
            tuple((pltpu.SemaphoreType.DMA @ vmesh) for _ in range(nbuf)),
            tuple((pltpu.SemaphoreType.DMA @ vmesh) for _ in range(nbuf)),
            (pltpu.SMEM @ smesh)((s_chunk,), jnp.int32),
            pltpu.VMEM_SHARED((2, s_chunk, D_MODEL), jnp.float32),
            pltpu.SemaphoreType.DMA @ smesh,
            pltpu.SemaphoreType.DMA @ smesh,
        ],
        compiler_params=pltpu.CompilerParams(skip_device_barrier=True),
    )


def kernel(toks, W_E):
    n_batch, seq = toks.shape
    flat = toks.reshape(n_batch * seq).astype(jnp.int32)
    out = _build_mpmd_kernel(
        n_batch, seq, tec_cols=3 * seq // 4,
        chunk=16, nbuf=4, lag=2, s_chunk=256,
    )(flat, W_E)
    return out.reshape(n_batch, seq, D_MODEL)
